# baseline (speedup 1.0000x reference)
import functools

import jax
import jax.numpy as jnp
from jax import lax
from jax.experimental import pallas as pl
from jax.experimental.pallas import tpu as pltpu
from jax.experimental.pallas import tpu_sc as plsc

EMBD = 64
NSETS = 6
BATCH = 4096
RTOT = NSETS * BATCH
NWORK = 32
ROWS_PER_W = RTOT // NWORK
CHUNK = 128
NCHUNK = ROWS_PER_W // CHUNK
TILE = 256
NTILES = RTOT // TILE
TPB = BATCH // TILE


def _sc_gather(user_W, poi_W, time_W, uidx3, pidx3, tidx3):
    mesh = plsc.VectorSubcoreMesh(core_axis_name="c", subcore_axis_name="s")
    row_ty = jax.ShapeDtypeStruct((RTOT, EMBD), jnp.float32)

    @functools.partial(
        pl.kernel,
        mesh=mesh,
        out_type=[row_ty, row_ty, row_ty],
        scratch_types=[
            pltpu.VMEM((NCHUNK, CHUNK), jnp.int32),
            pltpu.VMEM((ROWS_PER_W, EMBD), jnp.float32),
            pltpu.SemaphoreType.DMA,
        ],
        compiler_params=pltpu.CompilerParams(use_tc_tiling_on_sc=False),
    )
    def gk(user_hbm, poi_hbm, time_hbm, uidx_hbm, pidx_hbm, tidx_hbm,
           out_u, out_p, out_t, idx_v, rows_v, sem):
        wid = lax.axis_index("s") * 2 + lax.axis_index("c")
        base = wid * ROWS_PER_W
        for table, idxh, out in ((user_hbm, uidx_hbm, out_u),
                                 (poi_hbm, pidx_hbm, out_p),
                                 (time_hbm, tidx_hbm, out_t)):
            pltpu.sync_copy(idxh.at[wid], idx_v)
            copies = [
                pltpu.async_copy(table.at[idx_v.at[j]],
                                 rows_v.at[pl.ds(j * CHUNK, CHUNK)], sem)
                for j in range(NCHUNK)
            ]
            for c in copies:
                c.wait()
            pltpu.sync_copy(rows_v, out.at[pl.ds(base, ROWS_PER_W)])

    return gk(user_W, poi_W, time_W, uidx3, pidx3, tidx3)


def _score_body(tidx_ref, u_ref, p_ref, t_ref, proj_ref, out_ref, neg_ref):
    i = pl.program_id(0)
    s = i // TPB
    u = u_ref[...]
    p = p_ref[...]
    t = t_ref[...]
    tid = tidx_ref[0, 0, :]
    nt = proj_ref.shape[0]
    oh = (tid[:, None] == lax.broadcasted_iota(jnp.int32, (TILE, nt), 1))
    H = jnp.dot(oh.astype(jnp.float32), proj_ref[...],
                preferred_element_type=jnp.float32)
    H3 = H.reshape(TILE, EMBD, EMBD)
    v = jnp.sum(H3 * u[:, None, :], axis=2)
    score = jnp.sum((v + t) * p, axis=1)
    sp = jnp.log(1.0 + jnp.exp(-jnp.abs(score)))
    ls_pos = jnp.minimum(score, 0.0) - sp
    ls_neg = -jnp.maximum(score, 0.0) - sp
    out_ref[...] = (-ls_pos).reshape(1, TILE // 128, 128)

    @pl.when(i == TPB)
    def _init():
        neg_ref[...] = jnp.zeros_like(neg_ref)

    @pl.when(i >= TPB)
    def _acc():
        rows = lax.broadcasted_iota(jnp.int32, (8, 128), 0)
        cols = lax.broadcasted_iota(jnp.int32, (8, 128), 1)
        mask = (rows == (s - 1)) & (cols == 0)
        neg_ref[...] += jnp.where(mask, -jnp.sum(ls_neg), 0.0)


def _tc_score(tidx3, u_rows, p_rows, t_rows, proj_W):
    return pl.pallas_call(
        _score_body,
        grid=(NTILES,),
        in_specs=[
            pl.BlockSpec((1, 1, TILE), lambda i: (i, 0, 0)),
            pl.BlockSpec((TILE, EMBD), lambda i: (i, 0)),
            pl.BlockSpec((TILE, EMBD), lambda i: (i, 0)),
            pl.BlockSpec((TILE, EMBD), lambda i: (i, 0)),
            pl.BlockSpec(proj_W.shape, lambda i: (0, 0)),
        ],
        out_specs=[
            pl.BlockSpec((1, TILE // 128, 128), lambda i: (i, 0, 0)),
            pl.BlockSpec((8, 128), lambda i: (0, 0)),
        ],
        out_shape=[
            jax.ShapeDtypeStruct((NTILES, TILE // 128, 128), jnp.float32),
            jax.ShapeDtypeStruct((8, 128), jnp.float32),
        ],
    )(tidx3, u_rows, p_rows, t_rows, proj_W)


def kernel(pos_u, pos_t, pos_p, neg_u, neg_t, neg_p, NS, user_W, poi_W,
           time_W, proj_W):
    nneg = neg_u.shape[0]
    all_u = jnp.concatenate([pos_u[None, :], neg_u], 0).reshape(-1).astype(jnp.int32)
    all_t = jnp.concatenate([pos_t[None, :], neg_t], 0).reshape(-1).astype(jnp.int32)
    all_p = jnp.concatenate([pos_p[None, :], neg_p], 0).reshape(-1).astype(jnp.int32)
    u_rows, p_rows, t_rows = _sc_gather(
        user_W, poi_W, time_W,
        all_u.reshape(NWORK, NCHUNK, CHUNK),
        all_p.reshape(NWORK, NCHUNK, CHUNK),
        all_t.reshape(NWORK, NCHUNK, CHUNK))
    out_all, neg_out = _tc_score(all_t.reshape(NTILES, 1, TILE),
                                 u_rows, p_rows, t_rows, proj_W)
    pos = out_all.reshape(-1)[:BATCH]
    neg = neg_out[:nneg, 0]
    return (pos, neg)


# speedup vs baseline: 2.2047x; 2.2047x over previous
import functools

import jax
import jax.numpy as jnp
from jax import lax
from jax.experimental import pallas as pl
from jax.experimental.pallas import tpu as pltpu
from jax.experimental.pallas import tpu_sc as plsc

EMBD = 64
NSETS = 6
BATCH = 4096
RTOT = NSETS * BATCH
NWORK = 32
ROWS_PER_W = RTOT // NWORK
CHUNK = 128
NCHUNK = ROWS_PER_W // CHUNK
TILE = 256
NTILES = RTOT // TILE
TPB = BATCH // TILE


def _sc_gather(user2, poi2, uidx3, pidx3):
    mesh = plsc.VectorSubcoreMesh(core_axis_name="c", subcore_axis_name="s")
    row_ty = jax.ShapeDtypeStruct((RTOT, 2 * EMBD), jnp.float32)

    @functools.partial(
        pl.kernel,
        mesh=mesh,
        out_type=[row_ty, row_ty],
        scratch_types=[
            pltpu.VMEM((NCHUNK, CHUNK), jnp.int32),
            pltpu.VMEM((ROWS_PER_W, 2 * EMBD), jnp.float32),
            pltpu.SemaphoreType.DMA,
        ],
    )
    def gk(user_hbm, poi_hbm, uidx_hbm, pidx_hbm, out_u, out_p,
           idx_v, rows_v, sem):
        wid = lax.axis_index("s") * 2 + lax.axis_index("c")
        base = wid * ROWS_PER_W
        for table, idxh, out in ((user_hbm, uidx_hbm, out_u),
                                 (poi_hbm, pidx_hbm, out_p)):
            pltpu.sync_copy(idxh.at[wid], idx_v)
            copies = [
                pltpu.async_copy(table.at[idx_v.at[j]],
                                 rows_v.at[pl.ds(j * CHUNK, CHUNK)], sem)
                for j in range(NCHUNK)
            ]
            for c in copies:
                c.wait()
            pltpu.sync_copy(rows_v, out.at[pl.ds(base, ROWS_PER_W)])

    return gk(user2, poi2, uidx3, pidx3)


def _score_body(tidx_ref, uidx_ref, pidx_ref, uT_ref, pT_ref, projT_ref,
                timeT_ref, out_ref, neg_ref):
    i = pl.program_id(0)
    s = i // TPB
    nt = projT_ref.shape[1]
    tid = tidx_ref[0]
    iota_t = lax.broadcasted_iota(jnp.int32, (nt, TILE), 0)
    ohf = (tid == iota_t).astype(jnp.float32)
    HT = jnp.dot(projT_ref[...], ohf.astype(jnp.bfloat16),
                 preferred_element_type=jnp.float32)
    tsel = jnp.dot(timeT_ref[...], ohf,
                   preferred_element_type=jnp.float32)
    uhalf = (uidx_ref[0] & 1) == 1
    phalf = (pidx_ref[0] & 1) == 1
    urows = uT_ref[...]
    prows = pT_ref[...]
    u = jnp.where(uhalf, urows[EMBD:, :], urows[:EMBD, :])
    p = jnp.where(phalf, prows[EMBD:, :], prows[:EMBD, :])
    H3 = HT.reshape(EMBD, EMBD, TILE)
    v = jnp.sum(H3 * u[None, :, :], axis=1)
    score = jnp.sum((v + tsel) * p, axis=0)
    sp = jnp.log(1.0 + jnp.exp(-jnp.abs(score)))
    ls_pos = jnp.minimum(score, 0.0) - sp
    ls_neg = -jnp.maximum(score, 0.0) - sp
    out_ref[...] = (-ls_pos).reshape(1, 1, TILE)

    @pl.when(i == TPB)
    def _init():
        neg_ref[...] = jnp.zeros_like(neg_ref)

    @pl.when(i >= TPB)
    def _acc():
        rows = lax.broadcasted_iota(jnp.int32, (8, 128), 0)
        cols = lax.broadcasted_iota(jnp.int32, (8, 128), 1)
        mask = (rows == (s - 1)) & (cols == 0)
        neg_ref[...] += jnp.where(mask, -jnp.sum(ls_neg), 0.0)


def _tc_score(tidx3, uidx3, pidx3, uT, pT, projT_bf, timeT):
    return pl.pallas_call(
        _score_body,
        grid=(NTILES,),
        in_specs=[
            pl.BlockSpec((1, 1, TILE), lambda i: (i, 0, 0)),
            pl.BlockSpec((1, 1, TILE), lambda i: (i, 0, 0)),
            pl.BlockSpec((1, 1, TILE), lambda i: (i, 0, 0)),
            pl.BlockSpec((2 * EMBD, TILE), lambda i: (0, i)),
            pl.BlockSpec((2 * EMBD, TILE), lambda i: (0, i)),
            pl.BlockSpec(projT_bf.shape, lambda i: (0, 0)),
            pl.BlockSpec(timeT.shape, lambda i: (0, 0)),
        ],
        out_specs=[
            pl.BlockSpec((1, 1, TILE), lambda i: (i, 0, 0)),
            pl.BlockSpec((8, 128), lambda i: (0, 0)),
        ],
        out_shape=[
            jax.ShapeDtypeStruct((NTILES, 1, TILE), jnp.float32),
            jax.ShapeDtypeStruct((8, 128), jnp.float32),
        ],
    )(tidx3, uidx3, pidx3, uT, pT, projT_bf, timeT)


def kernel(pos_u, pos_t, pos_p, neg_u, neg_t, neg_p, NS, user_W, poi_W,
           time_W, proj_W):
    nneg = neg_u.shape[0]
    all_u = jnp.concatenate([pos_u[None, :], neg_u], 0).reshape(-1).astype(jnp.int32)
    all_t = jnp.concatenate([pos_t[None, :], neg_t], 0).reshape(-1).astype(jnp.int32)
    all_p = jnp.concatenate([pos_p[None, :], neg_p], 0).reshape(-1).astype(jnp.int32)
    user2 = user_W.reshape(-1, 2 * EMBD)
    poi2 = poi_W.reshape(-1, 2 * EMBD)
    u_pairs, p_pairs = _sc_gather(
        user2, poi2,
        (all_u >> 1).reshape(NWORK, NCHUNK, CHUNK),
        (all_p >> 1).reshape(NWORK, NCHUNK, CHUNK))
    out_all, neg_out = _tc_score(
        all_t.reshape(NTILES, 1, TILE),
        all_u.reshape(NTILES, 1, TILE),
        all_p.reshape(NTILES, 1, TILE),
        u_pairs.T, p_pairs.T,
        proj_W.T.astype(jnp.bfloat16),
        time_W.T)
    pos = out_all.reshape(-1)[:BATCH]
    neg = neg_out[:nneg, 0]
    return (pos, neg)


# speedup vs baseline: 2.2778x; 1.0332x over previous
import functools

import jax
import jax.numpy as jnp
from jax import lax
from jax.experimental import pallas as pl
from jax.experimental.pallas import tpu as pltpu
from jax.experimental.pallas import tpu_sc as plsc

EMBD = 64
NSETS = 6
BATCH = 4096
RTOT = NSETS * BATCH
NWORK = 32
ROWS_PER_W = RTOT // NWORK
CHUNK = 128
NCHUNK = ROWS_PER_W // CHUNK
TILE = 256
NTILES = RTOT // TILE
TPB = BATCH // TILE


def _sc_gather(user2, poi2, uidx3, pidx3):
    mesh = plsc.VectorSubcoreMesh(core_axis_name="c", subcore_axis_name="s")
    row_ty = jax.ShapeDtypeStruct((RTOT, 2 * EMBD), jnp.float32)

    @functools.partial(
        pl.kernel,
        mesh=mesh,
        out_type=[row_ty, row_ty],
        scratch_types=[
            pltpu.VMEM((NCHUNK, CHUNK), jnp.int32),
            pltpu.VMEM((ROWS_PER_W, 2 * EMBD), jnp.float32),
            pltpu.SemaphoreType.DMA,
        ],
    )
    def gk(user_hbm, poi_hbm, uidx_hbm, pidx_hbm, out_u, out_p,
           idx_v, rows_v, sem):
        wid = lax.axis_index("s") * 2 + lax.axis_index("c")
        base = wid * ROWS_PER_W
        for table, idxh, out in ((user_hbm, uidx_hbm, out_u),
                                 (poi_hbm, pidx_hbm, out_p)):
            pltpu.sync_copy(idxh.at[wid], idx_v)
            copies = [
                pltpu.async_copy(table.at[idx_v.at[j]],
                                 rows_v.at[pl.ds(j * CHUNK, CHUNK)], sem)
                for j in range(NCHUNK)
            ]
            for c in copies:
                c.wait()
            pltpu.sync_copy(rows_v, out.at[pl.ds(base, ROWS_PER_W)])

    return gk(user2, poi2, uidx3, pidx3)


def _score_body(tidx_ref, uidx_ref, pidx_ref, uT_ref, pT_ref, projT_ref,
                timeT_ref, out_ref, neg_ref):
    i = pl.program_id(0)
    s = i // TPB
    nt = projT_ref.shape[1]
    tid = tidx_ref[0]
    iota_t = lax.broadcasted_iota(jnp.int32, (nt, TILE), 0)
    ohf = (tid == iota_t).astype(jnp.float32)
    HT = jnp.dot(projT_ref[...], ohf.astype(jnp.bfloat16),
                 preferred_element_type=jnp.float32)
    tsel = jnp.dot(timeT_ref[...], ohf,
                   preferred_element_type=jnp.float32)
    uhalf = (uidx_ref[0] & 1) == 1
    phalf = (pidx_ref[0] & 1) == 1
    urows = uT_ref[...].T
    prows = pT_ref[...].T
    u = jnp.where(uhalf, urows[EMBD:, :], urows[:EMBD, :])
    p = jnp.where(phalf, prows[EMBD:, :], prows[:EMBD, :])
    H3 = HT.reshape(EMBD, EMBD, TILE)
    v = jnp.sum(H3 * u[None, :, :], axis=1)
    score = jnp.sum((v + tsel) * p, axis=0)
    sp = jnp.log(1.0 + jnp.exp(-jnp.abs(score)))
    ls_pos = jnp.minimum(score, 0.0) - sp
    ls_neg = -jnp.maximum(score, 0.0) - sp
    out_ref[...] = (-ls_pos).reshape(1, 1, TILE)

    @pl.when(i == TPB)
    def _init():
        neg_ref[...] = jnp.zeros_like(neg_ref)

    @pl.when(i >= TPB)
    def _acc():
        rows = lax.broadcasted_iota(jnp.int32, (8, 128), 0)
        cols = lax.broadcasted_iota(jnp.int32, (8, 128), 1)
        mask = (rows == (s - 1)) & (cols == 0)
        neg_ref[...] += jnp.where(mask, -jnp.sum(ls_neg), 0.0)


def _tc_score(tidx3, uidx3, pidx3, uT, pT, projT_bf, timeT):
    return pl.pallas_call(
        _score_body,
        grid=(NTILES,),
        in_specs=[
            pl.BlockSpec((1, 1, TILE), lambda i: (i, 0, 0)),
            pl.BlockSpec((1, 1, TILE), lambda i: (i, 0, 0)),
            pl.BlockSpec((1, 1, TILE), lambda i: (i, 0, 0)),
            pl.BlockSpec((TILE, 2 * EMBD), lambda i: (i, 0)),
            pl.BlockSpec((TILE, 2 * EMBD), lambda i: (i, 0)),
            pl.BlockSpec(projT_bf.shape, lambda i: (0, 0)),
            pl.BlockSpec(timeT.shape, lambda i: (0, 0)),
        ],
        out_specs=[
            pl.BlockSpec((1, 1, TILE), lambda i: (i, 0, 0)),
            pl.BlockSpec((8, 128), lambda i: (0, 0)),
        ],
        out_shape=[
            jax.ShapeDtypeStruct((NTILES, 1, TILE), jnp.float32),
            jax.ShapeDtypeStruct((8, 128), jnp.float32),
        ],
    )(tidx3, uidx3, pidx3, uT, pT, projT_bf, timeT)


def kernel(pos_u, pos_t, pos_p, neg_u, neg_t, neg_p, NS, user_W, poi_W,
           time_W, proj_W):
    nneg = neg_u.shape[0]
    all_u = jnp.concatenate([pos_u[None, :], neg_u], 0).reshape(-1).astype(jnp.int32)
    all_t = jnp.concatenate([pos_t[None, :], neg_t], 0).reshape(-1).astype(jnp.int32)
    all_p = jnp.concatenate([pos_p[None, :], neg_p], 0).reshape(-1).astype(jnp.int32)
    user2 = user_W.reshape(-1, 2 * EMBD)
    poi2 = poi_W.reshape(-1, 2 * EMBD)
    u_pairs, p_pairs = _sc_gather(
        user2, poi2,
        (all_u >> 1).reshape(NWORK, NCHUNK, CHUNK),
        (all_p >> 1).reshape(NWORK, NCHUNK, CHUNK))
    out_all, neg_out = _tc_score(
        all_t.reshape(NTILES, 1, TILE),
        all_u.reshape(NTILES, 1, TILE),
        all_p.reshape(NTILES, 1, TILE),
        u_pairs, p_pairs,
        proj_W.T.astype(jnp.bfloat16),
        time_W.T)
    pos = out_all.reshape(-1)[:BATCH]
    neg = neg_out[:nneg, 0]
    return (pos, neg)
